# baseline (speedup 1.0000x reference)
import functools

import jax
import jax.numpy as jnp
from jax import lax
from jax.experimental import pallas as pl
from jax.experimental.pallas import tpu as pltpu
from jax.experimental.pallas import tpu_sc as plsc

D_K = 64
NNZ = 26
NC = 2
NS = 16
NW = NC * NS
ROWS_PER_CHUNK = 4
IDX_PER_CHUNK = ROWS_PER_CHUNK * NNZ

B_U = 16384
B_V = 4096
BW_U = B_U // NW
BW_V = B_V // NW
CH_U = BW_U // ROWS_PER_CHUNK
CH_V = BW_V // ROWS_PER_CHUNK


def _make_kernel():
    mesh = plsc.VectorSubcoreMesh(core_axis_name="c", subcore_axis_name="s")

    @functools.partial(
        pl.kernel,
        out_type=(
            jax.ShapeDtypeStruct((B_U, D_K), jnp.float32),
            jax.ShapeDtypeStruct((B_V, D_K), jnp.float32),
        ),
        mesh=mesh,
        compiler_params=pltpu.CompilerParams(use_tc_tiling_on_sc=False),
        scratch_types=[
            pltpu.VMEM((CH_U, IDX_PER_CHUNK), jnp.int32),
            pltpu.VMEM((IDX_PER_CHUNK, D_K), jnp.float32),
            pltpu.VMEM((BW_U, D_K), jnp.float32),
            pltpu.SemaphoreType.DMA,
        ],
    )
    def fm2(u_hbm, v_hbm, wu_hbm, wv_hbm, p_hbm, q_hbm, idx_v, buf_v, out_v, sem):
        wid = lax.axis_index("s") * NC + lax.axis_index("c")

        def run_table(tbl_hbm, idx_hbm, out_hbm, n_chunks, bw):
            pltpu.sync_copy(idx_hbm.at[wid], idx_v.at[pl.ds(0, n_chunks)])

            def chunk_body(g, carry):
                pltpu.async_copy(tbl_hbm.at[idx_v.at[g]], buf_v, sem).wait()
                for r in range(ROWS_PER_CHUNK):
                    row = g * ROWS_PER_CHUNK + r
                    for v in range(ROWS_PER_CHUNK):
                        acc = buf_v[r * NNZ, pl.ds(v * 16, 16)]
                        for j in range(1, NNZ):
                            acc = acc + buf_v[r * NNZ + j, pl.ds(v * 16, 16)]
                        out_v[row, pl.ds(v * 16, 16)] = acc
                return carry

            lax.fori_loop(0, n_chunks, chunk_body, 0)
            pltpu.sync_copy(
                out_v.at[pl.ds(0, bw)], out_hbm.at[pl.ds(wid * bw, bw)]
            )

        run_table(wu_hbm, u_hbm, p_hbm, CH_U, BW_U)
        run_table(wv_hbm, v_hbm, q_hbm, CH_V, BW_V)

    return fm2


_FM2 = _make_kernel()


@jax.jit
def kernel(U, V, W_u, W_v):
    u_idx = U.astype(jnp.int32).reshape(NW, CH_U, IDX_PER_CHUNK)
    v_idx = V.astype(jnp.int32).reshape(NW, CH_V, IDX_PER_CHUNK)
    return _FM2(u_idx, v_idx, W_u, W_v)
